# baseline (speedup 1.0000x reference)
import jax
import jax.numpy as jnp
from jax import lax
from jax.experimental import pallas as pl
from jax.experimental.pallas import tpu as pltpu


def _copy_kernel(feat_ref, out_ref):
    out_ref[0] = feat_ref[0]


def kernel(features, lengths):
    b, c, t = features.shape
    cb = 512
    w = t + 16
    return pl.pallas_call(
        _copy_kernel,
        grid=(b, c // cb, 9),
        in_specs=[
            pl.BlockSpec((1, cb, 512), lambda i, j, k: (i, j, jnp.minimum(k, 7)))
        ],
        out_specs=pl.BlockSpec((1, cb, 512), lambda i, j, k: (i, j, k)),
        out_shape=jax.ShapeDtypeStruct((b, c, w), features.dtype),
    )(features)


# speedup vs baseline: 1.1826x; 1.1826x over previous
import jax
import jax.numpy as jnp
from jax import lax
from jax.experimental import pallas as pl
from jax.experimental.pallas import tpu as pltpu

LEFT = 8
RIGHT = 8
WIN = 272


def _pad_kernel(lengths_ref, left_ref, feat_ref, out_hbm, main_buf, tail_buf,
                sem_m, sem_t):
    nb, nc = pl.num_programs(0), pl.num_programs(1)
    b, cbk = pl.program_id(0), pl.program_id(1)
    i = b * nc + cbk
    n = nb * nc
    slot = lax.rem(i, 2)
    l = lengths_ref[b]
    feat = feat_ref[0]
    cb, t = feat.shape
    w = t + LEFT + RIGHT
    c0 = cbk * cb

    def body_copy(s):
        return pltpu.make_async_copy(
            main_buf.at[s], out_hbm.at[b, pl.ds(c0, cb), pl.ds(0, t)],
            sem_m.at[s])

    def tail_copy(s):
        return pltpu.make_async_copy(
            tail_buf.at[s], out_hbm.at[b, pl.ds(c0, cb), pl.ds(t, 16)],
            sem_t.at[s])

    @pl.when(i >= 2)
    def _():
        body_copy(slot).wait()
        tail_copy(slot).wait()

    left = left_ref[0]
    zeros = jnp.zeros((cb, RIGHT), feat.dtype)
    base = jnp.concatenate([left, feat, zeros], axis=-1)
    main_buf[slot] = base[:, :t]
    tail_buf[slot] = base[:, t:]

    a = pl.multiple_of(jnp.minimum(((l - 9) // 128) * 128, t - 256), 128)
    win = feat_ref[0, :, pl.ds(a, 256)]
    off = (l - 9) - a
    r1 = pltpu.roll(win, (256 - off) % 256, axis=1)
    s8 = r1[:, :8]
    strip = jnp.concatenate([s8[:, 7 - j:8 - j] for j in range(8)], axis=-1)
    ws = pl.multiple_of(jnp.minimum(((l + LEFT) // 128) * 128, w - WIN), 128)
    poff = (l + LEFT) - ws
    strip_pad = jnp.concatenate(
        [strip, jnp.zeros((cb, WIN - 8), feat.dtype)], axis=-1)
    placed = pltpu.roll(strip_pad, poff, axis=1)
    pos = lax.broadcasted_iota(jnp.int32, (cb, WIN), 1)
    mask = (pos >= poff) & (pos < poff + RIGHT)

    @pl.when(ws < w - WIN)
    def _():
        cur = main_buf[slot, :, pl.ds(ws, WIN)]
        main_buf[slot, :, pl.ds(ws, WIN)] = jnp.where(mask, placed, cur)

    @pl.when(ws == w - WIN)
    def _():
        wse = pl.multiple_of(w - WIN, 128)
        cur = main_buf[slot, :, pl.ds(wse, WIN - 16)]
        main_buf[slot, :, pl.ds(wse, WIN - 16)] = jnp.where(
            mask[:, :WIN - 16], placed[:, :WIN - 16], cur)
        cur_t = tail_buf[slot]
        tail_buf[slot] = jnp.where(mask[:, WIN - 16:], placed[:, WIN - 16:],
                                   cur_t)

    body_copy(slot).start()
    tail_copy(slot).start()

    @pl.when(i == n - 1)
    def _():
        body_copy(slot).wait()
        tail_copy(slot).wait()
        other = 1 - slot

        @pl.when(n >= 2)
        def _():
            body_copy(other).wait()
            tail_copy(other).wait()


def kernel(features, lengths):
    b, c, t = features.shape
    cb = 256
    left_src = lax.rev(
        lax.slice(features, (0, 0, 1), (1, c, 1 + LEFT)), (2,)
    )
    return pl.pallas_call(
        _pad_kernel,
        grid=(b, c // cb),
        in_specs=[
            pl.BlockSpec(memory_space=pltpu.SMEM),
            pl.BlockSpec((1, cb, LEFT), lambda i, j: (0, j, 0)),
            pl.BlockSpec((1, cb, t), lambda i, j: (i, j, 0)),
        ],
        out_specs=pl.BlockSpec(memory_space=pl.ANY),
        out_shape=jax.ShapeDtypeStruct((b, c, t + LEFT + RIGHT), features.dtype),
        scratch_shapes=[
            pltpu.VMEM((2, cb, t), features.dtype),
            pltpu.VMEM((2, cb, LEFT + RIGHT), features.dtype),
            pltpu.SemaphoreType.DMA((2,)),
            pltpu.SemaphoreType.DMA((2,)),
        ],
    )(lengths, left_src, features)
